# baseline (speedup 1.0000x reference)
import dataclasses
import functools

import jax
import jax.numpy as jnp
from jax import lax
from jax.experimental import pallas as pl
from jax.experimental.pallas import tpu as pltpu
from jax.experimental.pallas import tpu_sc as plsc

N = 10000
E = 320000
D = 128
DOUT = 200
NG = 8

NTILES = 32
EBLK = 128
NHALF = 40
NBLK = 2 * NHALF
EPT = NBLK * EBLK
EPAD = NTILES * EPT
RPT = 632
NPAD = 16 * RPT
DW = 16

BLK = 2000
GRID = N // BLK

_HI = lax.Precision.HIGHEST

_SC_CP = pltpu.CompilerParams()
if "needs_layout_passes" in pltpu.CompilerParams.__dataclass_fields__:
    _SC_CP = dataclasses.replace(_SC_CP, needs_layout_passes=False)


def _zero_rows(buf, nrows, width):
    @pl.loop(0, nrows)
    def _(r):
        for f in range(width // 16):
            buf[r, pl.ds(16 * f, 16)] = jnp.zeros((16,), jnp.float32)


_CHUNKS = tuple((64 * i, 64) for i in range(9)) + ((576, RPT - 576),)


def _sc_scatter(y, srcs, dsts, ews):
    mesh = plsc.VectorSubcoreMesh(core_axis_name="c", subcore_axis_name="s")

    @functools.partial(
        pl.kernel, mesh=mesh,
        out_type=jax.ShapeDtypeStruct((2, NPAD, D), jnp.float32),
        scratch_types=[
            pltpu.VMEM((NBLK, EBLK), jnp.int32),
            pltpu.VMEM((NBLK, EBLK), jnp.int32),
            pltpu.VMEM((EPT,), jnp.float32),
            pltpu.VMEM((EBLK, D), jnp.float32),
            pltpu.VMEM_SHARED((NPAD, D), jnp.float32),
        ],
        compiler_params=_SC_CP,
    )
    def pass_(y_hbm, src_hbm, dst_hbm, ew_hbm, out_hbm,
              src_v, dst_v, ew_v, r0, acc_sh):
        c = lax.axis_index("c")
        s = lax.axis_index("s")
        wid = c * 16 + s
        for h in range(2):
            pltpu.sync_copy(src_hbm.at[wid, h],
                            src_v.at[pl.ds(h * NHALF, NHALF)])
            pltpu.sync_copy(dst_hbm.at[wid, h],
                            dst_v.at[pl.ds(h * NHALF, NHALF)])
            pltpu.sync_copy(ew_hbm.at[wid, h],
                            ew_v.at[pl.ds(h * NHALF * EBLK, NHALF * EBLK)])

        _zero_rows(r0, EBLK, D)
        base = s * RPT
        for off, nr in _CHUNKS:
            pltpu.sync_copy(r0.at[pl.ds(0, nr)],
                            acc_sh.at[pl.ds(base + off, nr)])
        plsc.subcore_barrier()

        @pl.loop(0, NBLK)
        def _(b):
            pltpu.sync_copy(y_hbm.at[src_v.at[b]], r0)

            @pl.loop(0, EBLK)
            def _(j):
                idx = lax.broadcast_in_dim(b * EBLK + j, (16,), ())
                w16 = plsc.load_gather(ew_v, [idx])
                for f in range(D // 16):
                    sl = pl.ds(16 * f, 16)
                    r0[j, sl] = r0[j, sl] * w16

            pltpu.sync_copy(r0, acc_sh.at[dst_v.at[b]], add=True)

        plsc.subcore_barrier()
        for off, nr in _CHUNKS:
            pltpu.sync_copy(acc_sh.at[pl.ds(base + off, nr)],
                            out_hbm.at[c, pl.ds(base + off, nr)])

    return pass_(y, srcs, dsts, ews)


def _sc_degree(dsts, ews):
    mesh = plsc.VectorSubcoreMesh(core_axis_name="c", subcore_axis_name="s")

    @functools.partial(
        pl.kernel, mesh=mesh,
        out_type=jax.ShapeDtypeStruct((2, NPAD, D), jnp.float32),
        scratch_types=[
            pltpu.VMEM((NHALF, EBLK), jnp.int32),
            pltpu.VMEM((NHALF * EBLK,), jnp.float32),
            pltpu.VMEM((EBLK, D), jnp.float32),
            pltpu.VMEM_SHARED((NPAD, D), jnp.float32),
        ],
        compiler_params=_SC_CP,
    )
    def pass_(dst_hbm, ew_hbm, out_hbm, dst_v, ew_v, rows_v, acc_sh):
        c = lax.axis_index("c")
        s = lax.axis_index("s")
        wid = c * 16 + s

        _zero_rows(rows_v, EBLK, D)
        base = s * RPT
        for off, nr in _CHUNKS:
            pltpu.sync_copy(rows_v.at[pl.ds(0, nr)],
                            acc_sh.at[pl.ds(base + off, nr)])
        plsc.subcore_barrier()

        for h in range(2):
            pltpu.sync_copy(dst_hbm.at[wid, h], dst_v)
            pltpu.sync_copy(ew_hbm.at[wid, h], ew_v)

            @pl.loop(0, NHALF)
            def _(b):
                @pl.loop(0, EBLK, step=16)
                def _(g):
                    gbase = lax.broadcast_in_dim(b * EBLK + g, (16,), ())
                    for j in range(16):
                        rows_v[g + j, pl.ds(0, DW)] = plsc.load_gather(
                            ew_v, [gbase + j])

                pltpu.sync_copy(rows_v, acc_sh.at[dst_v.at[b]], add=True)

        plsc.subcore_barrier()
        for off, nr in _CHUNKS:
            pltpu.sync_copy(acc_sh.at[pl.ds(base + off, nr)],
                            out_hbm.at[c, pl.ds(base + off, nr)])

    return pass_(dsts, ews)


def _tc_matmul(x, W):
    def body(x_ref, w_ref, o_ref):
        o_ref[...] = lax.dot_general(x_ref[...], w_ref[...],
                                     (((1,), (0,)), ((), ())), precision=_HI)

    return pl.pallas_call(
        body,
        grid=(GRID,),
        in_specs=[pl.BlockSpec((BLK, D), lambda i: (i, 0)),
                  pl.BlockSpec((D, D), lambda i: (0, 0))],
        out_specs=pl.BlockSpec((BLK, D), lambda i: (i, 0)),
        out_shape=jax.ShapeDtypeStruct((N, D), jnp.float32),
    )(x, W)


def _tc_dinv_scale(t1, degp):
    def body(t_ref, d_ref, y_ref, dinv_ref):
        deg = 1.0 + d_ref[0, :, 0:1] + d_ref[1, :, 0:1]
        r = lax.rsqrt(deg)
        dinv = r * (1.5 - 0.5 * deg * r * r)
        dinv_ref[...] = dinv
        y_ref[...] = dinv * t_ref[...]

    return pl.pallas_call(
        body,
        grid=(GRID,),
        in_specs=[pl.BlockSpec((BLK, D), lambda i: (i, 0)),
                  pl.BlockSpec((2, BLK, D), lambda i: (0, i, 0))],
        out_specs=[pl.BlockSpec((BLK, D), lambda i: (i, 0)),
                   pl.BlockSpec((BLK, 1), lambda i: (i, 0))],
        out_shape=[jax.ShapeDtypeStruct((N, D), jnp.float32),
                   jax.ShapeDtypeStruct((N, 1), jnp.float32)],
    )(t1, degp)


def _tc_layer(acc, y, dinv, b, W):
    def body(a_ref, y_ref, di_ref, b_ref, w_ref, o_ref):
        di = di_ref[...]
        a = a_ref[0] + a_ref[1] + y_ref[...]
        h = jnp.maximum(di * a + b_ref[...], 0.0)
        o_ref[...] = di * lax.dot_general(h, w_ref[...],
                                          (((1,), (0,)), ((), ())),
                                          precision=_HI)

    return pl.pallas_call(
        body,
        grid=(GRID,),
        in_specs=[pl.BlockSpec((2, BLK, D), lambda i: (0, i, 0)),
                  pl.BlockSpec((BLK, D), lambda i: (i, 0)),
                  pl.BlockSpec((BLK, 1), lambda i: (i, 0)),
                  pl.BlockSpec((1, D), lambda i: (0, 0)),
                  pl.BlockSpec((D, D), lambda i: (0, 0))],
        out_specs=pl.BlockSpec((BLK, D), lambda i: (i, 0)),
        out_shape=jax.ShapeDtypeStruct((N, D), jnp.float32),
    )(acc, y, dinv, b, W)


def _tc_elem(acc, y, dinv, b):
    def body(a_ref, y_ref, di_ref, b_ref, o_ref):
        di = di_ref[...]
        a = a_ref[0] + a_ref[1] + y_ref[...]
        o_ref[...] = di * jnp.maximum(di * a + b_ref[...], 0.0)

    return pl.pallas_call(
        body,
        grid=(GRID,),
        in_specs=[pl.BlockSpec((2, BLK, D), lambda i: (0, i, 0)),
                  pl.BlockSpec((BLK, D), lambda i: (i, 0)),
                  pl.BlockSpec((BLK, 1), lambda i: (i, 0)),
                  pl.BlockSpec((1, D), lambda i: (0, 0))],
        out_specs=pl.BlockSpec((BLK, D), lambda i: (i, 0)),
        out_shape=jax.ShapeDtypeStruct((N, D), jnp.float32),
    )(acc, y, dinv, b)


def _tc_final(acc, y, dinv, batch2, W3, b3, Wp, bp):
    def body(a_ref, y_ref, di_ref, bt_ref, w3_ref, b3_ref, wp_ref, bp_ref,
             o_ref, sums, cnt):
        i = pl.program_id(0)

        @pl.when(i == 0)
        def _():
            sums[...] = jnp.zeros((NG, D), jnp.float32)
            cnt[...] = jnp.zeros((NG, 1), jnp.float32)

        z = di_ref[...] * (a_ref[0] + a_ref[1] + y_ref[...])
        gids = lax.broadcasted_iota(jnp.int32, (NG, BLK), 0)
        mask = (gids == bt_ref[...][:, 0][None, :]).astype(jnp.float32)
        sums[...] += lax.dot_general(mask, z, (((1,), (0,)), ((), ())),
                                     precision=_HI)
        cnt[...] += jnp.sum(mask, axis=1, keepdims=True)

        @pl.when(i == GRID - 1)
        def _():
            c = cnt[...]
            pooled = sums[...] / jnp.maximum(c, 1.0)
            t = lax.dot_general(pooled, w3_ref[...],
                                (((1,), (0,)), ((), ())), precision=_HI)
            t = jnp.where(c > 0.0, t + b3_ref[...], 0.0)
            o_ref[...] = lax.dot_general(t, wp_ref[...],
                                         (((1,), (0,)), ((), ())),
                                         precision=_HI) + bp_ref[...]

    return pl.pallas_call(
        body,
        grid=(GRID,),
        in_specs=[pl.BlockSpec((2, BLK, D), lambda i: (0, i, 0)),
                  pl.BlockSpec((BLK, D), lambda i: (i, 0)),
                  pl.BlockSpec((BLK, 1), lambda i: (i, 0)),
                  pl.BlockSpec((BLK, 1), lambda i: (i, 0)),
                  pl.BlockSpec((D, DOUT), lambda i: (0, 0)),
                  pl.BlockSpec((1, DOUT), lambda i: (0, 0)),
                  pl.BlockSpec((DOUT, 4), lambda i: (0, 0)),
                  pl.BlockSpec((1, 4), lambda i: (0, 0))],
        out_specs=pl.BlockSpec((NG, 4), lambda i: (0, 0)),
        out_shape=jax.ShapeDtypeStruct((NG, 4), jnp.float32),
        scratch_shapes=[pltpu.VMEM((NG, D), jnp.float32),
                        pltpu.VMEM((NG, 1), jnp.float32)],
    )(acc, y, dinv, batch2, W3, b3, Wp, bp)


def kernel(x, edge_index, edge_attr, batch, W1, b1, W2, b2, W3, b3, Wp, bp):
    src = edge_index[0].astype(jnp.int32)
    dst = edge_index[1].astype(jnp.int32)
    ew = edge_attr.astype(jnp.float32)
    pad = EPAD - E
    srcs = jnp.pad(src, (0, pad)).reshape(NTILES, 2, NHALF, EBLK)
    dsts = jnp.pad(dst, (0, pad)).reshape(NTILES, 2, NHALF, EBLK)
    ews = jnp.pad(ew, (0, pad)).reshape(NTILES, 2, NHALF * EBLK)
    batch2 = batch.astype(jnp.int32).reshape(N, 1)
    b1r, b2r = b1.reshape(1, D), b2.reshape(1, D)
    b3r, bpr = b3.reshape(1, DOUT), bp.reshape(1, 4)

    degp = _sc_degree(dsts, ews)[:, :N]
    t1 = _tc_matmul(x, W1)
    y1, dinv = _tc_dinv_scale(t1, degp)
    acc1 = _sc_scatter(y1, srcs, dsts, ews)[:, :N]
    y2 = _tc_layer(acc1, y1, dinv, b1r, W2)
    acc2 = _sc_scatter(y2, srcs, dsts, ews)[:, :N]
    y3 = _tc_elem(acc2, y2, dinv, b2r)
    acc3 = _sc_scatter(y3, srcs, dsts, ews)[:, :N]
    return _tc_final(acc3, y3, dinv, batch2, W3, b3r, Wp, bpr)


# speedup vs baseline: 1.4309x; 1.4309x over previous
import dataclasses
import functools

import jax
import jax.numpy as jnp
from jax import lax
from jax.experimental import pallas as pl
from jax.experimental.pallas import tpu as pltpu
from jax.experimental.pallas import tpu_sc as plsc

N = 10000
E = 320000
D = 128
DOUT = 200
NG = 8

NTILES = 32
EBLK = 128
NBLK = 79
EPT = NBLK * EBLK
EPAD = NTILES * EPT
RPT = 632
NPAD = 16 * RPT
DW = 16

BLK = 2000
GRID = N // BLK

_HI = lax.Precision.HIGHEST

_SC_CP = pltpu.CompilerParams()
if "needs_layout_passes" in pltpu.CompilerParams.__dataclass_fields__:
    _SC_CP = dataclasses.replace(_SC_CP, needs_layout_passes=False)


def _zero_rows(buf, nrows, width):
    @pl.loop(0, nrows)
    def _(r):
        for f in range(width // 16):
            buf[r, pl.ds(16 * f, 16)] = jnp.zeros((16,), jnp.float32)


_CHUNKS = ((0, 128), (128, 128), (256, 128), (384, 128), (512, RPT - 512))


def _sc_scatter(y, srcs, dsts, ews):
    mesh = plsc.VectorSubcoreMesh(core_axis_name="c", subcore_axis_name="s")

    @functools.partial(
        pl.kernel, mesh=mesh,
        out_type=jax.ShapeDtypeStruct((2, NPAD, D), jnp.float32),
        scratch_types=[
            pltpu.VMEM((NBLK, EBLK), jnp.int32),
            pltpu.VMEM((NBLK, EBLK), jnp.int32),
            pltpu.VMEM((EPT,), jnp.float32),
            pltpu.VMEM((EBLK, D), jnp.float32),
            pltpu.VMEM_SHARED((NPAD, D), jnp.float32),
        ],
        compiler_params=_SC_CP,
    )
    def pass_(y_hbm, src_hbm, dst_hbm, ew_hbm, out_hbm,
              src_v, dst_v, ew_v, r0, acc_sh):
        c = lax.axis_index("c")
        s = lax.axis_index("s")
        wid = c * 16 + s
        pltpu.sync_copy(src_hbm.at[wid], src_v)
        pltpu.sync_copy(dst_hbm.at[wid], dst_v)
        pltpu.sync_copy(ew_hbm.at[wid], ew_v)

        _zero_rows(r0, EBLK, D)
        base = s * RPT
        for off, nr in _CHUNKS:
            pltpu.sync_copy(r0.at[pl.ds(0, nr)],
                            acc_sh.at[pl.ds(base + off, nr)])
        plsc.subcore_barrier()

        @pl.loop(0, NBLK)
        def _(b):
            pltpu.sync_copy(y_hbm.at[src_v.at[b]], r0)

            @pl.loop(0, EBLK)
            def _(j):
                idx = lax.broadcast_in_dim(b * EBLK + j, (16,), ())
                w16 = plsc.load_gather(ew_v, [idx])
                for f in range(D // 16):
                    sl = pl.ds(16 * f, 16)
                    r0[j, sl] = r0[j, sl] * w16

            pltpu.sync_copy(r0, acc_sh.at[dst_v.at[b]], add=True)

        plsc.subcore_barrier()
        for off, nr in _CHUNKS:
            pltpu.sync_copy(acc_sh.at[pl.ds(base + off, nr)],
                            out_hbm.at[c, pl.ds(base + off, nr)])

    return pass_(y, srcs, dsts, ews)


def _sc_degree(dsts, ews):
    mesh = plsc.VectorSubcoreMesh(core_axis_name="c", subcore_axis_name="s")

    @functools.partial(
        pl.kernel, mesh=mesh,
        out_type=jax.ShapeDtypeStruct((2, NPAD, D), jnp.float32),
        scratch_types=[
            pltpu.VMEM((NBLK, EBLK), jnp.int32),
            pltpu.VMEM((EPT,), jnp.float32),
            pltpu.VMEM((EBLK, D), jnp.float32),
            pltpu.VMEM_SHARED((NPAD, D), jnp.float32),
        ],
        compiler_params=_SC_CP,
    )
    def pass_(dst_hbm, ew_hbm, out_hbm, dst_v, ew_v, rows_v, acc_sh):
        c = lax.axis_index("c")
        s = lax.axis_index("s")
        wid = c * 16 + s

        _zero_rows(rows_v, EBLK, D)
        base = s * RPT
        for off, nr in _CHUNKS:
            pltpu.sync_copy(rows_v.at[pl.ds(0, nr)],
                            acc_sh.at[pl.ds(base + off, nr)])
        plsc.subcore_barrier()

        pltpu.sync_copy(dst_hbm.at[wid], dst_v)
        pltpu.sync_copy(ew_hbm.at[wid], ew_v)

        @pl.loop(0, NBLK)
        def _(b):
            @pl.loop(0, EBLK)
            def _(j):
                idx = lax.broadcast_in_dim(b * EBLK + j, (16,), ())
                rows_v[j, pl.ds(0, DW)] = plsc.load_gather(ew_v, [idx])

            pltpu.sync_copy(rows_v, acc_sh.at[dst_v.at[b]], add=True)

        plsc.subcore_barrier()
        for off, nr in _CHUNKS:
            pltpu.sync_copy(acc_sh.at[pl.ds(base + off, nr)],
                            out_hbm.at[c, pl.ds(base + off, nr)])

    return pass_(dsts, ews)


def _tc_matmul(x, W):
    def body(x_ref, w_ref, o_ref):
        o_ref[...] = lax.dot_general(x_ref[...], w_ref[...],
                                     (((1,), (0,)), ((), ())), precision=_HI)

    return pl.pallas_call(
        body,
        grid=(GRID,),
        in_specs=[pl.BlockSpec((BLK, D), lambda i: (i, 0)),
                  pl.BlockSpec((D, D), lambda i: (0, 0))],
        out_specs=pl.BlockSpec((BLK, D), lambda i: (i, 0)),
        out_shape=jax.ShapeDtypeStruct((N, D), jnp.float32),
    )(x, W)


def _tc_dinv_scale(t1, degp):
    def body(t_ref, d_ref, y_ref, dinv_ref):
        deg = 1.0 + d_ref[0, :, 0:1] + d_ref[1, :, 0:1]
        r = lax.rsqrt(deg)
        dinv = r * (1.5 - 0.5 * deg * r * r)
        dinv_ref[...] = dinv
        y_ref[...] = dinv * t_ref[...]

    return pl.pallas_call(
        body,
        grid=(GRID,),
        in_specs=[pl.BlockSpec((BLK, D), lambda i: (i, 0)),
                  pl.BlockSpec((2, BLK, D), lambda i: (0, i, 0))],
        out_specs=[pl.BlockSpec((BLK, D), lambda i: (i, 0)),
                   pl.BlockSpec((BLK, 1), lambda i: (i, 0))],
        out_shape=[jax.ShapeDtypeStruct((N, D), jnp.float32),
                   jax.ShapeDtypeStruct((N, 1), jnp.float32)],
    )(t1, degp)


def _tc_layer(acc, y, dinv, b, W):
    def body(a_ref, y_ref, di_ref, b_ref, w_ref, o_ref):
        di = di_ref[...]
        a = a_ref[0] + a_ref[1] + y_ref[...]
        h = jnp.maximum(di * a + b_ref[...], 0.0)
        o_ref[...] = di * lax.dot_general(h, w_ref[...],
                                          (((1,), (0,)), ((), ())),
                                          precision=_HI)

    return pl.pallas_call(
        body,
        grid=(GRID,),
        in_specs=[pl.BlockSpec((2, BLK, D), lambda i: (0, i, 0)),
                  pl.BlockSpec((BLK, D), lambda i: (i, 0)),
                  pl.BlockSpec((BLK, 1), lambda i: (i, 0)),
                  pl.BlockSpec((1, D), lambda i: (0, 0)),
                  pl.BlockSpec((D, D), lambda i: (0, 0))],
        out_specs=pl.BlockSpec((BLK, D), lambda i: (i, 0)),
        out_shape=jax.ShapeDtypeStruct((N, D), jnp.float32),
    )(acc, y, dinv, b, W)


def _tc_elem(acc, y, dinv, b):
    def body(a_ref, y_ref, di_ref, b_ref, o_ref):
        di = di_ref[...]
        a = a_ref[0] + a_ref[1] + y_ref[...]
        o_ref[...] = di * jnp.maximum(di * a + b_ref[...], 0.0)

    return pl.pallas_call(
        body,
        grid=(GRID,),
        in_specs=[pl.BlockSpec((2, BLK, D), lambda i: (0, i, 0)),
                  pl.BlockSpec((BLK, D), lambda i: (i, 0)),
                  pl.BlockSpec((BLK, 1), lambda i: (i, 0)),
                  pl.BlockSpec((1, D), lambda i: (0, 0))],
        out_specs=pl.BlockSpec((BLK, D), lambda i: (i, 0)),
        out_shape=jax.ShapeDtypeStruct((N, D), jnp.float32),
    )(acc, y, dinv, b)


def _tc_final(acc, y, dinv, batch2, W3, b3, Wp, bp):
    def body(a_ref, y_ref, di_ref, bt_ref, w3_ref, b3_ref, wp_ref, bp_ref,
             o_ref, sums, cnt):
        i = pl.program_id(0)

        @pl.when(i == 0)
        def _():
            sums[...] = jnp.zeros((NG, D), jnp.float32)
            cnt[...] = jnp.zeros((NG, 1), jnp.float32)

        z = di_ref[...] * (a_ref[0] + a_ref[1] + y_ref[...])
        gids = lax.broadcasted_iota(jnp.int32, (NG, BLK), 0)
        mask = (gids == bt_ref[...][:, 0][None, :]).astype(jnp.float32)
        sums[...] += lax.dot_general(mask, z, (((1,), (0,)), ((), ())),
                                     precision=_HI)
        cnt[...] += jnp.sum(mask, axis=1, keepdims=True)

        @pl.when(i == GRID - 1)
        def _():
            c = cnt[...]
            pooled = sums[...] / jnp.maximum(c, 1.0)
            t = lax.dot_general(pooled, w3_ref[...],
                                (((1,), (0,)), ((), ())), precision=_HI)
            t = jnp.where(c > 0.0, t + b3_ref[...], 0.0)
            o_ref[...] = lax.dot_general(t, wp_ref[...],
                                         (((1,), (0,)), ((), ())),
                                         precision=_HI) + bp_ref[...]

    return pl.pallas_call(
        body,
        grid=(GRID,),
        in_specs=[pl.BlockSpec((2, BLK, D), lambda i: (0, i, 0)),
                  pl.BlockSpec((BLK, D), lambda i: (i, 0)),
                  pl.BlockSpec((BLK, 1), lambda i: (i, 0)),
                  pl.BlockSpec((BLK, 1), lambda i: (i, 0)),
                  pl.BlockSpec((D, DOUT), lambda i: (0, 0)),
                  pl.BlockSpec((1, DOUT), lambda i: (0, 0)),
                  pl.BlockSpec((DOUT, 4), lambda i: (0, 0)),
                  pl.BlockSpec((1, 4), lambda i: (0, 0))],
        out_specs=pl.BlockSpec((NG, 4), lambda i: (0, 0)),
        out_shape=jax.ShapeDtypeStruct((NG, 4), jnp.float32),
        scratch_shapes=[pltpu.VMEM((NG, D), jnp.float32),
                        pltpu.VMEM((NG, 1), jnp.float32)],
    )(acc, y, dinv, batch2, W3, b3, Wp, bp)


def kernel(x, edge_index, edge_attr, batch, W1, b1, W2, b2, W3, b3, Wp, bp):
    src = edge_index[0].astype(jnp.int32)
    dst = edge_index[1].astype(jnp.int32)
    ew = edge_attr.astype(jnp.float32)
    pad = EPAD - E
    srcs = jnp.pad(src, (0, pad)).reshape(NTILES, NBLK, EBLK)
    dsts = jnp.pad(dst, (0, pad)).reshape(NTILES, NBLK, EBLK)
    ews = jnp.pad(ew, (0, pad)).reshape(NTILES, EPT)
    batch2 = batch.astype(jnp.int32).reshape(N, 1)
    b1r, b2r = b1.reshape(1, D), b2.reshape(1, D)
    b3r, bpr = b3.reshape(1, DOUT), bp.reshape(1, 4)

    degp = _sc_degree(dsts, ews)[:, :N]
    t1 = _tc_matmul(x, W1)
    y1, dinv = _tc_dinv_scale(t1, degp)
    acc1 = _sc_scatter(y1, srcs, dsts, ews)[:, :N]
    y2 = _tc_layer(acc1, y1, dinv, b1r, W2)
    acc2 = _sc_scatter(y2, srcs, dsts, ews)[:, :N]
    y3 = _tc_elem(acc2, y2, dinv, b2r)
    acc3 = _sc_scatter(y3, srcs, dsts, ews)[:, :N]
    return _tc_final(acc3, y3, dinv, batch2, W3, b3r, Wp, bpr)


# speedup vs baseline: 1.4609x; 1.0210x over previous
import dataclasses
import functools

import jax
import jax.numpy as jnp
from jax import lax
from jax.experimental import pallas as pl
from jax.experimental.pallas import tpu as pltpu
from jax.experimental.pallas import tpu_sc as plsc

N = 10000
E = 320000
D = 128
DOUT = 200
NG = 8

NTILES = 32
EBLK = 128
NBLK = 79
EPT = NBLK * EBLK
EPAD = NTILES * EPT
RPT = 632
NPAD = 16 * RPT
DW = 16

BLK = 2000
GRID = N // BLK

_HI = lax.Precision.HIGHEST

_SC_CP = pltpu.CompilerParams()
if "needs_layout_passes" in pltpu.CompilerParams.__dataclass_fields__:
    _SC_CP = dataclasses.replace(_SC_CP, needs_layout_passes=False)


def _zero_rows(buf, nrows, width):
    @pl.loop(0, nrows)
    def _(r):
        for f in range(width // 16):
            buf[r, pl.ds(16 * f, 16)] = jnp.zeros((16,), jnp.float32)


_CHUNKS = ((0, 128), (128, 128), (256, 128), (384, 128), (512, RPT - 512))


def _sc_scatter(y, srcs, dsts, ews):
    mesh = plsc.VectorSubcoreMesh(core_axis_name="c", subcore_axis_name="s")

    @functools.partial(
        pl.kernel, mesh=mesh,
        out_type=jax.ShapeDtypeStruct((2, NPAD, D), jnp.float32),
        scratch_types=[
            pltpu.VMEM((NBLK, EBLK), jnp.int32),
            pltpu.VMEM((NBLK, EBLK), jnp.int32),
            pltpu.VMEM((EPT,), jnp.float32),
            pltpu.VMEM((EBLK, D), jnp.float32),
            pltpu.VMEM_SHARED((NPAD, D), jnp.float32),
        ],
        compiler_params=_SC_CP,
    )
    def pass_(y_hbm, src_hbm, dst_hbm, ew_hbm, out_hbm,
              src_v, dst_v, ew_v, r0, acc_sh):
        c = lax.axis_index("c")
        s = lax.axis_index("s")
        wid = c * 16 + s
        pltpu.sync_copy(src_hbm.at[wid], src_v)
        pltpu.sync_copy(dst_hbm.at[wid], dst_v)
        pltpu.sync_copy(ew_hbm.at[wid], ew_v)

        _zero_rows(r0, EBLK, D)
        base = s * RPT
        for off, nr in _CHUNKS:
            pltpu.sync_copy(r0.at[pl.ds(0, nr)],
                            acc_sh.at[pl.ds(base + off, nr)])
        plsc.subcore_barrier()

        @pl.loop(0, NBLK)
        def _(b):
            pltpu.sync_copy(y_hbm.at[src_v.at[b]], r0)

            @pl.loop(0, EBLK, step=8)
            def _(g):
                gbase = lax.broadcast_in_dim(b * EBLK + g, (16,), ())
                for j in range(8):
                    wj = plsc.load_gather(ew_v, [gbase + j])
                    for f in range(D // 16):
                        sl = pl.ds(16 * f, 16)
                        r0[g + j, sl] = r0[g + j, sl] * wj

            pltpu.sync_copy(r0, acc_sh.at[dst_v.at[b]], add=True)

        plsc.subcore_barrier()
        for off, nr in _CHUNKS:
            pltpu.sync_copy(acc_sh.at[pl.ds(base + off, nr)],
                            out_hbm.at[c, pl.ds(base + off, nr)])

    return pass_(y, srcs, dsts, ews)


def _sc_degree(dsts, ews):
    mesh = plsc.VectorSubcoreMesh(core_axis_name="c", subcore_axis_name="s")

    @functools.partial(
        pl.kernel, mesh=mesh,
        out_type=jax.ShapeDtypeStruct((2, NPAD, D), jnp.float32),
        scratch_types=[
            pltpu.VMEM((NBLK, EBLK), jnp.int32),
            pltpu.VMEM((EPT,), jnp.float32),
            pltpu.VMEM((EBLK, D), jnp.float32),
            pltpu.VMEM_SHARED((NPAD, D), jnp.float32),
        ],
        compiler_params=_SC_CP,
    )
    def pass_(dst_hbm, ew_hbm, out_hbm, dst_v, ew_v, rows_v, acc_sh):
        c = lax.axis_index("c")
        s = lax.axis_index("s")
        wid = c * 16 + s

        _zero_rows(rows_v, EBLK, D)
        base = s * RPT
        for off, nr in _CHUNKS:
            pltpu.sync_copy(rows_v.at[pl.ds(0, nr)],
                            acc_sh.at[pl.ds(base + off, nr)])
        plsc.subcore_barrier()

        pltpu.sync_copy(dst_hbm.at[wid], dst_v)
        pltpu.sync_copy(ew_hbm.at[wid], ew_v)

        @pl.loop(0, NBLK)
        def _(b):
            @pl.loop(0, EBLK)
            def _(j):
                idx = lax.broadcast_in_dim(b * EBLK + j, (16,), ())
                rows_v[j, pl.ds(0, DW)] = plsc.load_gather(ew_v, [idx])

            pltpu.sync_copy(rows_v, acc_sh.at[dst_v.at[b]], add=True)

        plsc.subcore_barrier()
        for off, nr in _CHUNKS:
            pltpu.sync_copy(acc_sh.at[pl.ds(base + off, nr)],
                            out_hbm.at[c, pl.ds(base + off, nr)])

    return pass_(dsts, ews)


def _tc_matmul(x, W):
    def body(x_ref, w_ref, o_ref):
        o_ref[...] = lax.dot_general(x_ref[...], w_ref[...],
                                     (((1,), (0,)), ((), ())), precision=_HI)

    return pl.pallas_call(
        body,
        grid=(GRID,),
        in_specs=[pl.BlockSpec((BLK, D), lambda i: (i, 0)),
                  pl.BlockSpec((D, D), lambda i: (0, 0))],
        out_specs=pl.BlockSpec((BLK, D), lambda i: (i, 0)),
        out_shape=jax.ShapeDtypeStruct((N, D), jnp.float32),
    )(x, W)


def _tc_dinv_scale(t1, degp):
    def body(t_ref, d_ref, y_ref, dinv_ref):
        deg = 1.0 + d_ref[0, :, 0:1] + d_ref[1, :, 0:1]
        r = lax.rsqrt(deg)
        dinv = r * (1.5 - 0.5 * deg * r * r)
        dinv_ref[...] = dinv
        y_ref[...] = dinv * t_ref[...]

    return pl.pallas_call(
        body,
        grid=(GRID,),
        in_specs=[pl.BlockSpec((BLK, D), lambda i: (i, 0)),
                  pl.BlockSpec((2, BLK, D), lambda i: (0, i, 0))],
        out_specs=[pl.BlockSpec((BLK, D), lambda i: (i, 0)),
                   pl.BlockSpec((BLK, 1), lambda i: (i, 0))],
        out_shape=[jax.ShapeDtypeStruct((N, D), jnp.float32),
                   jax.ShapeDtypeStruct((N, 1), jnp.float32)],
    )(t1, degp)


def _tc_layer(acc, y, dinv, b, W):
    def body(a_ref, y_ref, di_ref, b_ref, w_ref, o_ref):
        di = di_ref[...]
        a = a_ref[0] + a_ref[1] + y_ref[...]
        h = jnp.maximum(di * a + b_ref[...], 0.0)
        o_ref[...] = di * lax.dot_general(h, w_ref[...],
                                          (((1,), (0,)), ((), ())),
                                          precision=_HI)

    return pl.pallas_call(
        body,
        grid=(GRID,),
        in_specs=[pl.BlockSpec((2, BLK, D), lambda i: (0, i, 0)),
                  pl.BlockSpec((BLK, D), lambda i: (i, 0)),
                  pl.BlockSpec((BLK, 1), lambda i: (i, 0)),
                  pl.BlockSpec((1, D), lambda i: (0, 0)),
                  pl.BlockSpec((D, D), lambda i: (0, 0))],
        out_specs=pl.BlockSpec((BLK, D), lambda i: (i, 0)),
        out_shape=jax.ShapeDtypeStruct((N, D), jnp.float32),
    )(acc, y, dinv, b, W)


def _tc_elem(acc, y, dinv, b):
    def body(a_ref, y_ref, di_ref, b_ref, o_ref):
        di = di_ref[...]
        a = a_ref[0] + a_ref[1] + y_ref[...]
        o_ref[...] = di * jnp.maximum(di * a + b_ref[...], 0.0)

    return pl.pallas_call(
        body,
        grid=(GRID,),
        in_specs=[pl.BlockSpec((2, BLK, D), lambda i: (0, i, 0)),
                  pl.BlockSpec((BLK, D), lambda i: (i, 0)),
                  pl.BlockSpec((BLK, 1), lambda i: (i, 0)),
                  pl.BlockSpec((1, D), lambda i: (0, 0))],
        out_specs=pl.BlockSpec((BLK, D), lambda i: (i, 0)),
        out_shape=jax.ShapeDtypeStruct((N, D), jnp.float32),
    )(acc, y, dinv, b)


def _tc_final(acc, y, dinv, batch2, W3, b3, Wp, bp):
    def body(a_ref, y_ref, di_ref, bt_ref, w3_ref, b3_ref, wp_ref, bp_ref,
             o_ref, sums, cnt):
        i = pl.program_id(0)

        @pl.when(i == 0)
        def _():
            sums[...] = jnp.zeros((NG, D), jnp.float32)
            cnt[...] = jnp.zeros((NG, 1), jnp.float32)

        z = di_ref[...] * (a_ref[0] + a_ref[1] + y_ref[...])
        gids = lax.broadcasted_iota(jnp.int32, (NG, BLK), 0)
        mask = (gids == bt_ref[...][:, 0][None, :]).astype(jnp.float32)
        sums[...] += lax.dot_general(mask, z, (((1,), (0,)), ((), ())),
                                     precision=_HI)
        cnt[...] += jnp.sum(mask, axis=1, keepdims=True)

        @pl.when(i == GRID - 1)
        def _():
            c = cnt[...]
            pooled = sums[...] / jnp.maximum(c, 1.0)
            t = lax.dot_general(pooled, w3_ref[...],
                                (((1,), (0,)), ((), ())), precision=_HI)
            t = jnp.where(c > 0.0, t + b3_ref[...], 0.0)
            o_ref[...] = lax.dot_general(t, wp_ref[...],
                                         (((1,), (0,)), ((), ())),
                                         precision=_HI) + bp_ref[...]

    return pl.pallas_call(
        body,
        grid=(GRID,),
        in_specs=[pl.BlockSpec((2, BLK, D), lambda i: (0, i, 0)),
                  pl.BlockSpec((BLK, D), lambda i: (i, 0)),
                  pl.BlockSpec((BLK, 1), lambda i: (i, 0)),
                  pl.BlockSpec((BLK, 1), lambda i: (i, 0)),
                  pl.BlockSpec((D, DOUT), lambda i: (0, 0)),
                  pl.BlockSpec((1, DOUT), lambda i: (0, 0)),
                  pl.BlockSpec((DOUT, 4), lambda i: (0, 0)),
                  pl.BlockSpec((1, 4), lambda i: (0, 0))],
        out_specs=pl.BlockSpec((NG, 4), lambda i: (0, 0)),
        out_shape=jax.ShapeDtypeStruct((NG, 4), jnp.float32),
        scratch_shapes=[pltpu.VMEM((NG, D), jnp.float32),
                        pltpu.VMEM((NG, 1), jnp.float32)],
    )(acc, y, dinv, batch2, W3, b3, Wp, bp)


def kernel(x, edge_index, edge_attr, batch, W1, b1, W2, b2, W3, b3, Wp, bp):
    src = edge_index[0].astype(jnp.int32)
    dst = edge_index[1].astype(jnp.int32)
    ew = edge_attr.astype(jnp.float32)
    pad = EPAD - E
    srcs = jnp.pad(src, (0, pad)).reshape(NTILES, NBLK, EBLK)
    dsts = jnp.pad(dst, (0, pad)).reshape(NTILES, NBLK, EBLK)
    ews = jnp.pad(ew, (0, pad)).reshape(NTILES, EPT)
    batch2 = batch.astype(jnp.int32).reshape(N, 1)
    b1r, b2r = b1.reshape(1, D), b2.reshape(1, D)
    b3r, bpr = b3.reshape(1, DOUT), bp.reshape(1, 4)

    degp = _sc_degree(dsts, ews)[:, :N]
    t1 = _tc_matmul(x, W1)
    y1, dinv = _tc_dinv_scale(t1, degp)
    acc1 = _sc_scatter(y1, srcs, dsts, ews)[:, :N]
    y2 = _tc_layer(acc1, y1, dinv, b1r, W2)
    acc2 = _sc_scatter(y2, srcs, dsts, ews)[:, :N]
    y3 = _tc_elem(acc2, y2, dinv, b2r)
    acc3 = _sc_scatter(y3, srcs, dsts, ews)[:, :N]
    return _tc_final(acc3, y3, dinv, batch2, W3, b3r, Wp, bpr)


# speedup vs baseline: 1.7351x; 1.1877x over previous
import dataclasses
import functools

import jax
import jax.numpy as jnp
from jax import lax
from jax.experimental import pallas as pl
from jax.experimental.pallas import tpu as pltpu
from jax.experimental.pallas import tpu_sc as plsc

N = 10000
E = 320000
D = 128
DOUT = 200
NG = 8

NTILES = 32
EBLK = 128
NBLK = 79
NHALF = 40
EPT = NBLK * EBLK
EPAD = NTILES * EPT
RPT = 632
NPAD = 16 * RPT
DW = 16

BLK = 2000
GRID = N // BLK

_HI = lax.Precision.HIGHEST

_SC_CP = pltpu.CompilerParams()
if "needs_layout_passes" in pltpu.CompilerParams.__dataclass_fields__:
    _SC_CP = dataclasses.replace(_SC_CP, needs_layout_passes=False)


def _zero_rows(buf, nrows, width):
    @pl.loop(0, nrows)
    def _(r):
        for f in range(width // 16):
            buf[r, pl.ds(16 * f, 16)] = jnp.zeros((16,), jnp.float32)


_CHUNKS = ((0, 128), (128, 128), (256, 128), (384, 128), (512, RPT - 512))


def _sc_scatter(y, srcs, dsts, ews):
    mesh = plsc.VectorSubcoreMesh(core_axis_name="c", subcore_axis_name="s")

    @functools.partial(
        pl.kernel, mesh=mesh,
        out_type=jax.ShapeDtypeStruct((2, NPAD, D), jnp.float32),
        scratch_types=[
            pltpu.VMEM((NBLK, EBLK), jnp.int32),
            pltpu.VMEM((2, EBLK), jnp.int32),
            pltpu.VMEM((NHALF * EBLK,), jnp.float32),
            pltpu.VMEM((EBLK, D), jnp.float32),
            pltpu.VMEM((EBLK, D), jnp.float32),
            pltpu.VMEM_SHARED((NPAD, D), jnp.float32),
            pltpu.SemaphoreType.DMA((2,)),
            pltpu.SemaphoreType.DMA((2,)),
        ],
        compiler_params=_SC_CP,
    )
    def pass_(y_hbm, src_hbm, dst_hbm, ew_hbm, out_hbm,
              src_v, dstg, ew_v, r0, r1, acc_sh, gsem, dsem):
        bufs = (r0, r1)
        c = lax.axis_index("c")
        s = lax.axis_index("s")
        wid = c * 16 + s
        pltpu.sync_copy(src_hbm.at[wid], src_v)

        _zero_rows(r0, EBLK, D)
        base = s * RPT
        for off, nr in _CHUNKS:
            pltpu.sync_copy(r0.at[pl.ds(0, nr)],
                            acc_sh.at[pl.ds(base + off, nr)])
        plsc.subcore_barrier()

        def gather(b, k):
            return pltpu.make_async_copy(y_hbm.at[src_v.at[b]], bufs[k],
                                         gsem.at[k])

        def dstcp(b, k):
            return pltpu.make_async_copy(
                dst_hbm.at[pl.ds(wid * EPT + b * EBLK, EBLK)], dstg.at[k],
                dsem.at[k])

        def mul(b, lb, k):
            buf = bufs[k]

            @pl.loop(0, EBLK, step=8)
            def _(g):
                gbase = lax.broadcast_in_dim(lb * EBLK + g, (16,), ())
                for j in range(8):
                    wj = plsc.load_gather(ew_v, [gbase + j])
                    for f in range(D // 16):
                        sl = pl.ds(16 * f, 16)
                        buf[g + j, sl] = buf[g + j, sl] * wj

        def block(b, lb, k, prefetch=True):
            gather(b, k).wait()
            dstcp(b, k).wait()
            if prefetch:
                gather(b + 1, 1 - k).start()
                dstcp(b + 1, 1 - k).start()
            mul(b, lb, k)
            pltpu.sync_copy(bufs[k], acc_sh.at[dstg.at[k]], add=True)

        gather(0, 0).start()
        dstcp(0, 0).start()
        for h, nb in ((0, NHALF), (1, NBLK - NHALF - 1)):
            pltpu.sync_copy(ew_hbm.at[wid, h], ew_v)

            @pl.loop(0, nb, step=2)
            def _(lb0):
                for k in range(2):
                    lb = lb0 + k
                    block(h * NHALF + lb, lb, k)

        block(NBLK - 1, NBLK - 1 - NHALF, 0, prefetch=False)

        plsc.subcore_barrier()
        for off, nr in _CHUNKS:
            pltpu.sync_copy(acc_sh.at[pl.ds(base + off, nr)],
                            out_hbm.at[c, pl.ds(base + off, nr)])

    return pass_(y, srcs, dsts, ews)


def _sc_degree(dsts, ews):
    mesh = plsc.VectorSubcoreMesh(core_axis_name="c", subcore_axis_name="s")

    @functools.partial(
        pl.kernel, mesh=mesh,
        out_type=jax.ShapeDtypeStruct((2, NPAD, D), jnp.float32),
        scratch_types=[
            pltpu.VMEM((NBLK, EBLK), jnp.int32),
            pltpu.VMEM((EPT,), jnp.float32),
            pltpu.VMEM((EBLK, D), jnp.float32),
            pltpu.VMEM_SHARED((NPAD, D), jnp.float32),
        ],
        compiler_params=_SC_CP,
    )
    def pass_(dst_hbm, ew_hbm, out_hbm, dst_v, ew_v, rows_v, acc_sh):
        c = lax.axis_index("c")
        s = lax.axis_index("s")
        wid = c * 16 + s

        _zero_rows(rows_v, EBLK, D)
        base = s * RPT
        for off, nr in _CHUNKS:
            pltpu.sync_copy(rows_v.at[pl.ds(0, nr)],
                            acc_sh.at[pl.ds(base + off, nr)])
        plsc.subcore_barrier()

        pltpu.sync_copy(dst_hbm.at[wid], dst_v)
        pltpu.sync_copy(ew_hbm.at[wid], ew_v)

        @pl.loop(0, NBLK)
        def _(b):
            @pl.loop(0, EBLK)
            def _(j):
                idx = lax.broadcast_in_dim(b * EBLK + j, (16,), ())
                rows_v[j, pl.ds(0, DW)] = plsc.load_gather(ew_v, [idx])

            pltpu.sync_copy(rows_v, acc_sh.at[dst_v.at[b]], add=True)

        plsc.subcore_barrier()
        for off, nr in _CHUNKS:
            pltpu.sync_copy(acc_sh.at[pl.ds(base + off, nr)],
                            out_hbm.at[c, pl.ds(base + off, nr)])

    return pass_(dsts, ews)


def _tc_matmul(x, W):
    def body(x_ref, w_ref, o_ref):
        o_ref[...] = lax.dot_general(x_ref[...], w_ref[...],
                                     (((1,), (0,)), ((), ())), precision=_HI)

    return pl.pallas_call(
        body,
        grid=(GRID,),
        in_specs=[pl.BlockSpec((BLK, D), lambda i: (i, 0)),
                  pl.BlockSpec((D, D), lambda i: (0, 0))],
        out_specs=pl.BlockSpec((BLK, D), lambda i: (i, 0)),
        out_shape=jax.ShapeDtypeStruct((N, D), jnp.float32),
    )(x, W)


def _tc_dinv_scale(t1, degp):
    def body(t_ref, d_ref, y_ref, dinv_ref):
        deg = 1.0 + d_ref[0, :, 0:1] + d_ref[1, :, 0:1]
        r = lax.rsqrt(deg)
        dinv = r * (1.5 - 0.5 * deg * r * r)
        dinv_ref[...] = dinv
        y_ref[...] = dinv * t_ref[...]

    return pl.pallas_call(
        body,
        grid=(GRID,),
        in_specs=[pl.BlockSpec((BLK, D), lambda i: (i, 0)),
                  pl.BlockSpec((2, BLK, D), lambda i: (0, i, 0))],
        out_specs=[pl.BlockSpec((BLK, D), lambda i: (i, 0)),
                   pl.BlockSpec((BLK, 1), lambda i: (i, 0))],
        out_shape=[jax.ShapeDtypeStruct((N, D), jnp.float32),
                   jax.ShapeDtypeStruct((N, 1), jnp.float32)],
    )(t1, degp)


def _tc_layer(acc, y, dinv, b, W):
    def body(a_ref, y_ref, di_ref, b_ref, w_ref, o_ref):
        di = di_ref[...]
        a = a_ref[0] + a_ref[1] + y_ref[...]
        h = jnp.maximum(di * a + b_ref[...], 0.0)
        o_ref[...] = di * lax.dot_general(h, w_ref[...],
                                          (((1,), (0,)), ((), ())),
                                          precision=_HI)

    return pl.pallas_call(
        body,
        grid=(GRID,),
        in_specs=[pl.BlockSpec((2, BLK, D), lambda i: (0, i, 0)),
                  pl.BlockSpec((BLK, D), lambda i: (i, 0)),
                  pl.BlockSpec((BLK, 1), lambda i: (i, 0)),
                  pl.BlockSpec((1, D), lambda i: (0, 0)),
                  pl.BlockSpec((D, D), lambda i: (0, 0))],
        out_specs=pl.BlockSpec((BLK, D), lambda i: (i, 0)),
        out_shape=jax.ShapeDtypeStruct((N, D), jnp.float32),
    )(acc, y, dinv, b, W)


def _tc_elem(acc, y, dinv, b):
    def body(a_ref, y_ref, di_ref, b_ref, o_ref):
        di = di_ref[...]
        a = a_ref[0] + a_ref[1] + y_ref[...]
        o_ref[...] = di * jnp.maximum(di * a + b_ref[...], 0.0)

    return pl.pallas_call(
        body,
        grid=(GRID,),
        in_specs=[pl.BlockSpec((2, BLK, D), lambda i: (0, i, 0)),
                  pl.BlockSpec((BLK, D), lambda i: (i, 0)),
                  pl.BlockSpec((BLK, 1), lambda i: (i, 0)),
                  pl.BlockSpec((1, D), lambda i: (0, 0))],
        out_specs=pl.BlockSpec((BLK, D), lambda i: (i, 0)),
        out_shape=jax.ShapeDtypeStruct((N, D), jnp.float32),
    )(acc, y, dinv, b)


def _tc_final(acc, y, dinv, batch2, W3, b3, Wp, bp):
    def body(a_ref, y_ref, di_ref, bt_ref, w3_ref, b3_ref, wp_ref, bp_ref,
             o_ref, sums, cnt):
        i = pl.program_id(0)

        @pl.when(i == 0)
        def _():
            sums[...] = jnp.zeros((NG, D), jnp.float32)
            cnt[...] = jnp.zeros((NG, 1), jnp.float32)

        z = di_ref[...] * (a_ref[0] + a_ref[1] + y_ref[...])
        gids = lax.broadcasted_iota(jnp.int32, (NG, BLK), 0)
        mask = (gids == bt_ref[...][:, 0][None, :]).astype(jnp.float32)
        sums[...] += lax.dot_general(mask, z, (((1,), (0,)), ((), ())),
                                     precision=_HI)
        cnt[...] += jnp.sum(mask, axis=1, keepdims=True)

        @pl.when(i == GRID - 1)
        def _():
            c = cnt[...]
            pooled = sums[...] / jnp.maximum(c, 1.0)
            t = lax.dot_general(pooled, w3_ref[...],
                                (((1,), (0,)), ((), ())), precision=_HI)
            t = jnp.where(c > 0.0, t + b3_ref[...], 0.0)
            o_ref[...] = lax.dot_general(t, wp_ref[...],
                                         (((1,), (0,)), ((), ())),
                                         precision=_HI) + bp_ref[...]

    return pl.pallas_call(
        body,
        grid=(GRID,),
        in_specs=[pl.BlockSpec((2, BLK, D), lambda i: (0, i, 0)),
                  pl.BlockSpec((BLK, D), lambda i: (i, 0)),
                  pl.BlockSpec((BLK, 1), lambda i: (i, 0)),
                  pl.BlockSpec((BLK, 1), lambda i: (i, 0)),
                  pl.BlockSpec((D, DOUT), lambda i: (0, 0)),
                  pl.BlockSpec((1, DOUT), lambda i: (0, 0)),
                  pl.BlockSpec((DOUT, 4), lambda i: (0, 0)),
                  pl.BlockSpec((1, 4), lambda i: (0, 0))],
        out_specs=pl.BlockSpec((NG, 4), lambda i: (0, 0)),
        out_shape=jax.ShapeDtypeStruct((NG, 4), jnp.float32),
        scratch_shapes=[pltpu.VMEM((NG, D), jnp.float32),
                        pltpu.VMEM((NG, 1), jnp.float32)],
    )(acc, y, dinv, batch2, W3, b3, Wp, bp)


def kernel(x, edge_index, edge_attr, batch, W1, b1, W2, b2, W3, b3, Wp, bp):
    src = edge_index[0].astype(jnp.int32)
    dst = edge_index[1].astype(jnp.int32)
    ew = edge_attr.astype(jnp.float32)
    pad = EPAD - E
    srcs = jnp.pad(src, (0, pad)).reshape(NTILES, NBLK, EBLK)
    dsts = jnp.pad(dst, (0, pad)).reshape(NTILES, NBLK, EBLK)
    ews = jnp.pad(jnp.pad(ew, (0, pad)).reshape(NTILES, EPT),
                  ((0, 0), (0, 2 * NHALF * EBLK - EPT)))
    ews = ews.reshape(NTILES, 2, NHALF * EBLK)
    ews_flat = ews.reshape(NTILES, 2 * NHALF * EBLK)[:, :EPT]
    batch2 = batch.astype(jnp.int32).reshape(N, 1)
    b1r, b2r = b1.reshape(1, D), b2.reshape(1, D)
    b3r, bpr = b3.reshape(1, DOUT), bp.reshape(1, 4)

    dsts_flat = jnp.pad(dst, (0, pad))
    degp = _sc_degree(dsts, ews_flat)[:, :N]
    t1 = _tc_matmul(x, W1)
    y1, dinv = _tc_dinv_scale(t1, degp)
    acc1 = _sc_scatter(y1, srcs, dsts_flat, ews)[:, :N]
    y2 = _tc_layer(acc1, y1, dinv, b1r, W2)
    acc2 = _sc_scatter(y2, srcs, dsts_flat, ews)[:, :N]
    y3 = _tc_elem(acc2, y2, dinv, b2r)
    acc3 = _sc_scatter(y3, srcs, dsts_flat, ews)[:, :N]
    return _tc_final(acc3, y3, dinv, batch2, W3, b3r, Wp, bpr)


# speedup vs baseline: 1.7474x; 1.0071x over previous
import dataclasses
import functools

import jax
import jax.numpy as jnp
from jax import lax
from jax.experimental import pallas as pl
from jax.experimental.pallas import tpu as pltpu
from jax.experimental.pallas import tpu_sc as plsc

N = 10000
E = 320000
D = 128
DOUT = 200
NG = 8

NTILES = 32
EBLK = 128
NBLK = 79
NHALF = 40
EPT = NBLK * EBLK
EPAD = NTILES * EPT
RPT = 632
NPAD = 16 * RPT
DW = 16

BLK = 2000
GRID = N // BLK

_HI = lax.Precision.HIGHEST

_SC_CP = pltpu.CompilerParams()
if "needs_layout_passes" in pltpu.CompilerParams.__dataclass_fields__:
    _SC_CP = dataclasses.replace(_SC_CP, needs_layout_passes=False)


def _zero_rows(buf, nrows, width):
    @pl.loop(0, nrows)
    def _(r):
        for f in range(width // 16):
            buf[r, pl.ds(16 * f, 16)] = jnp.zeros((16,), jnp.float32)


_CHUNKS = ((0, 128), (128, 128), (256, 128), (384, 128), (512, RPT - 512))


def _sc_scatter(y, srcs, dsts, ews):
    mesh = plsc.VectorSubcoreMesh(core_axis_name="c", subcore_axis_name="s")

    @functools.partial(
        pl.kernel, mesh=mesh,
        out_type=jax.ShapeDtypeStruct((2, NPAD, D), jnp.float32),
        scratch_types=[
            pltpu.VMEM((NBLK, EBLK), jnp.int32),
            pltpu.VMEM((2, EBLK), jnp.int32),
            pltpu.VMEM((NHALF * EBLK,), jnp.float32),
            pltpu.VMEM((EBLK, D), jnp.float32),
            pltpu.VMEM((EBLK, D), jnp.float32),
            pltpu.VMEM_SHARED((NPAD, D), jnp.float32),
            pltpu.SemaphoreType.DMA((2,)),
            pltpu.SemaphoreType.DMA((2,)),
        ],
        compiler_params=_SC_CP,
    )
    def pass_(y_hbm, src_hbm, dst_hbm, ew_hbm, out_hbm,
              src_v, dstg, ew_v, r0, r1, acc_sh, gsem, dsem):
        bufs = (r0, r1)
        c = lax.axis_index("c")
        s = lax.axis_index("s")
        wid = c * 16 + s
        pltpu.sync_copy(src_hbm.at[wid], src_v)

        _zero_rows(r0, EBLK, D)
        base = s * RPT
        for off, nr in _CHUNKS:
            pltpu.sync_copy(r0.at[pl.ds(0, nr)],
                            acc_sh.at[pl.ds(base + off, nr)])
        plsc.subcore_barrier()

        def gather(b, k):
            return pltpu.make_async_copy(y_hbm.at[src_v.at[b]], bufs[k],
                                         gsem.at[k])

        def dstcp(b, k):
            return pltpu.make_async_copy(
                dst_hbm.at[pl.ds(wid * EPT + b * EBLK, EBLK)], dstg.at[k],
                dsem.at[k])

        def mul(b, lb, k):
            buf = bufs[k]

            @pl.loop(0, EBLK, step=16)
            def _(g):
                gbase = lax.broadcast_in_dim(lb * EBLK + g, (16,), ())
                for j in range(16):
                    wj = plsc.load_gather(ew_v, [gbase + j])
                    for f in range(D // 16):
                        sl = pl.ds(16 * f, 16)
                        buf[g + j, sl] = buf[g + j, sl] * wj

        def block(b, lb, k, prefetch=True):
            gather(b, k).wait()
            dstcp(b, k).wait()
            if prefetch:
                gather(b + 1, 1 - k).start()
                dstcp(b + 1, 1 - k).start()
            mul(b, lb, k)
            pltpu.sync_copy(bufs[k], acc_sh.at[dstg.at[k]], add=True)

        gather(0, 0).start()
        dstcp(0, 0).start()
        for h, nb in ((0, NHALF), (1, NBLK - NHALF - 1)):
            pltpu.sync_copy(ew_hbm.at[wid, h], ew_v)

            @pl.loop(0, nb, step=2)
            def _(lb0):
                for k in range(2):
                    lb = lb0 + k
                    block(h * NHALF + lb, lb, k)

        block(NBLK - 1, NBLK - 1 - NHALF, 0, prefetch=False)

        plsc.subcore_barrier()
        for off, nr in _CHUNKS:
            pltpu.sync_copy(acc_sh.at[pl.ds(base + off, nr)],
                            out_hbm.at[c, pl.ds(base + off, nr)])

    return pass_(y, srcs, dsts, ews)


def _sc_degree(dsts, ews):
    mesh = plsc.VectorSubcoreMesh(core_axis_name="c", subcore_axis_name="s")

    @functools.partial(
        pl.kernel, mesh=mesh,
        out_type=jax.ShapeDtypeStruct((2, NPAD, D), jnp.float32),
        scratch_types=[
            pltpu.VMEM((NBLK, EBLK), jnp.int32),
            pltpu.VMEM((EPT,), jnp.float32),
            pltpu.VMEM((EBLK, D), jnp.float32),
            pltpu.VMEM_SHARED((NPAD, D), jnp.float32),
        ],
        compiler_params=_SC_CP,
    )
    def pass_(dst_hbm, ew_hbm, out_hbm, dst_v, ew_v, rows_v, acc_sh):
        c = lax.axis_index("c")
        s = lax.axis_index("s")
        wid = c * 16 + s

        _zero_rows(rows_v, EBLK, D)
        base = s * RPT
        for off, nr in _CHUNKS:
            pltpu.sync_copy(rows_v.at[pl.ds(0, nr)],
                            acc_sh.at[pl.ds(base + off, nr)])
        plsc.subcore_barrier()

        pltpu.sync_copy(dst_hbm.at[wid], dst_v)
        pltpu.sync_copy(ew_hbm.at[wid], ew_v)

        @pl.loop(0, NBLK)
        def _(b):
            @pl.loop(0, EBLK)
            def _(j):
                idx = lax.broadcast_in_dim(b * EBLK + j, (16,), ())
                rows_v[j, pl.ds(0, DW)] = plsc.load_gather(ew_v, [idx])

            pltpu.sync_copy(rows_v, acc_sh.at[dst_v.at[b]], add=True)

        plsc.subcore_barrier()
        for off, nr in _CHUNKS:
            pltpu.sync_copy(acc_sh.at[pl.ds(base + off, nr)],
                            out_hbm.at[c, pl.ds(base + off, nr)])

    return pass_(dsts, ews)


def _tc_matmul(x, W):
    def body(x_ref, w_ref, o_ref):
        o_ref[...] = lax.dot_general(x_ref[...], w_ref[...],
                                     (((1,), (0,)), ((), ())), precision=_HI)

    return pl.pallas_call(
        body,
        grid=(GRID,),
        in_specs=[pl.BlockSpec((BLK, D), lambda i: (i, 0)),
                  pl.BlockSpec((D, D), lambda i: (0, 0))],
        out_specs=pl.BlockSpec((BLK, D), lambda i: (i, 0)),
        out_shape=jax.ShapeDtypeStruct((N, D), jnp.float32),
    )(x, W)


def _tc_dinv_scale(t1, degp):
    def body(t_ref, d_ref, y_ref, dinv_ref):
        deg = 1.0 + d_ref[0, :, 0:1] + d_ref[1, :, 0:1]
        r = lax.rsqrt(deg)
        dinv = r * (1.5 - 0.5 * deg * r * r)
        dinv_ref[...] = dinv
        y_ref[...] = dinv * t_ref[...]

    return pl.pallas_call(
        body,
        grid=(GRID,),
        in_specs=[pl.BlockSpec((BLK, D), lambda i: (i, 0)),
                  pl.BlockSpec((2, BLK, D), lambda i: (0, i, 0))],
        out_specs=[pl.BlockSpec((BLK, D), lambda i: (i, 0)),
                   pl.BlockSpec((BLK, 1), lambda i: (i, 0))],
        out_shape=[jax.ShapeDtypeStruct((N, D), jnp.float32),
                   jax.ShapeDtypeStruct((N, 1), jnp.float32)],
    )(t1, degp)


def _tc_layer(acc, y, dinv, b, W):
    def body(a_ref, y_ref, di_ref, b_ref, w_ref, o_ref):
        di = di_ref[...]
        a = a_ref[0] + a_ref[1] + y_ref[...]
        h = jnp.maximum(di * a + b_ref[...], 0.0)
        o_ref[...] = di * lax.dot_general(h, w_ref[...],
                                          (((1,), (0,)), ((), ())),
                                          precision=_HI)

    return pl.pallas_call(
        body,
        grid=(GRID,),
        in_specs=[pl.BlockSpec((2, BLK, D), lambda i: (0, i, 0)),
                  pl.BlockSpec((BLK, D), lambda i: (i, 0)),
                  pl.BlockSpec((BLK, 1), lambda i: (i, 0)),
                  pl.BlockSpec((1, D), lambda i: (0, 0)),
                  pl.BlockSpec((D, D), lambda i: (0, 0))],
        out_specs=pl.BlockSpec((BLK, D), lambda i: (i, 0)),
        out_shape=jax.ShapeDtypeStruct((N, D), jnp.float32),
    )(acc, y, dinv, b, W)


def _tc_elem(acc, y, dinv, b):
    def body(a_ref, y_ref, di_ref, b_ref, o_ref):
        di = di_ref[...]
        a = a_ref[0] + a_ref[1] + y_ref[...]
        o_ref[...] = di * jnp.maximum(di * a + b_ref[...], 0.0)

    return pl.pallas_call(
        body,
        grid=(GRID,),
        in_specs=[pl.BlockSpec((2, BLK, D), lambda i: (0, i, 0)),
                  pl.BlockSpec((BLK, D), lambda i: (i, 0)),
                  pl.BlockSpec((BLK, 1), lambda i: (i, 0)),
                  pl.BlockSpec((1, D), lambda i: (0, 0))],
        out_specs=pl.BlockSpec((BLK, D), lambda i: (i, 0)),
        out_shape=jax.ShapeDtypeStruct((N, D), jnp.float32),
    )(acc, y, dinv, b)


def _tc_final(acc, y, dinv, batch2, W3, b3, Wp, bp):
    def body(a_ref, y_ref, di_ref, bt_ref, w3_ref, b3_ref, wp_ref, bp_ref,
             o_ref, sums, cnt):
        i = pl.program_id(0)

        @pl.when(i == 0)
        def _():
            sums[...] = jnp.zeros((NG, D), jnp.float32)
            cnt[...] = jnp.zeros((NG, 1), jnp.float32)

        z = di_ref[...] * (a_ref[0] + a_ref[1] + y_ref[...])
        gids = lax.broadcasted_iota(jnp.int32, (NG, BLK), 0)
        mask = (gids == bt_ref[...][:, 0][None, :]).astype(jnp.float32)
        sums[...] += lax.dot_general(mask, z, (((1,), (0,)), ((), ())),
                                     precision=_HI)
        cnt[...] += jnp.sum(mask, axis=1, keepdims=True)

        @pl.when(i == GRID - 1)
        def _():
            c = cnt[...]
            pooled = sums[...] / jnp.maximum(c, 1.0)
            t = lax.dot_general(pooled, w3_ref[...],
                                (((1,), (0,)), ((), ())), precision=_HI)
            t = jnp.where(c > 0.0, t + b3_ref[...], 0.0)
            o_ref[...] = lax.dot_general(t, wp_ref[...],
                                         (((1,), (0,)), ((), ())),
                                         precision=_HI) + bp_ref[...]

    return pl.pallas_call(
        body,
        grid=(GRID,),
        in_specs=[pl.BlockSpec((2, BLK, D), lambda i: (0, i, 0)),
                  pl.BlockSpec((BLK, D), lambda i: (i, 0)),
                  pl.BlockSpec((BLK, 1), lambda i: (i, 0)),
                  pl.BlockSpec((BLK, 1), lambda i: (i, 0)),
                  pl.BlockSpec((D, DOUT), lambda i: (0, 0)),
                  pl.BlockSpec((1, DOUT), lambda i: (0, 0)),
                  pl.BlockSpec((DOUT, 4), lambda i: (0, 0)),
                  pl.BlockSpec((1, 4), lambda i: (0, 0))],
        out_specs=pl.BlockSpec((NG, 4), lambda i: (0, 0)),
        out_shape=jax.ShapeDtypeStruct((NG, 4), jnp.float32),
        scratch_shapes=[pltpu.VMEM((NG, D), jnp.float32),
                        pltpu.VMEM((NG, 1), jnp.float32)],
    )(acc, y, dinv, batch2, W3, b3, Wp, bp)


def kernel(x, edge_index, edge_attr, batch, W1, b1, W2, b2, W3, b3, Wp, bp):
    src = edge_index[0].astype(jnp.int32)
    dst = edge_index[1].astype(jnp.int32)
    ew = edge_attr.astype(jnp.float32)
    pad = EPAD - E
    srcs = jnp.pad(src, (0, pad)).reshape(NTILES, NBLK, EBLK)
    dsts = jnp.pad(dst, (0, pad)).reshape(NTILES, NBLK, EBLK)
    ews = jnp.pad(jnp.pad(ew, (0, pad)).reshape(NTILES, EPT),
                  ((0, 0), (0, 2 * NHALF * EBLK - EPT)))
    ews = ews.reshape(NTILES, 2, NHALF * EBLK)
    ews_flat = ews.reshape(NTILES, 2 * NHALF * EBLK)[:, :EPT]
    batch2 = batch.astype(jnp.int32).reshape(N, 1)
    b1r, b2r = b1.reshape(1, D), b2.reshape(1, D)
    b3r, bpr = b3.reshape(1, DOUT), bp.reshape(1, 4)

    dsts_flat = jnp.pad(dst, (0, pad))
    degp = _sc_degree(dsts, ews_flat)[:, :N]
    t1 = _tc_matmul(x, W1)
    y1, dinv = _tc_dinv_scale(t1, degp)
    acc1 = _sc_scatter(y1, srcs, dsts_flat, ews)[:, :N]
    y2 = _tc_layer(acc1, y1, dinv, b1r, W2)
    acc2 = _sc_scatter(y2, srcs, dsts_flat, ews)[:, :N]
    y3 = _tc_elem(acc2, y2, dinv, b2r)
    acc3 = _sc_scatter(y3, srcs, dsts_flat, ews)[:, :N]
    return _tc_final(acc3, y3, dinv, batch2, W3, b3r, Wp, bpr)
